# Initial kernel scaffold; baseline (speedup 1.0000x reference)
import jax
import jax.numpy as jnp
from jax.experimental import pallas as pl


def kernel(x, params):
    raise NotImplementedError("write your pallas kernel here")


# speedup vs baseline: 5.4420x; 5.4420x over previous
import functools

import jax
import jax.numpy as jnp
from jax import lax
from jax.experimental import pallas as pl
from jax.experimental.pallas import tpu as pltpu
from jax.experimental.pallas import tpu_sc as plsc

F32 = jnp.float32
I32 = jnp.int32

NHEADS = 12
D = 768
DH = 64
DHID = 2048
E = 64
S = 2048
CAP = 128
NSLOT = E * CAP
TRASH = NSLOT
SLOT_ROWS = NSLOT + 8
RB = 256

_pcall = pl.pallas_call


BF16 = jnp.bfloat16


def _bdot(a, b, dims):
    return lax.dot_general(a.astype(BF16), b.astype(BF16), (dims, ((), ())),
                           preferred_element_type=F32)


def _qkv_body(x_ref, w_ref, b_ref, o_ref):
    o_ref[...] = _bdot(x_ref[...], w_ref[...], ((1,), (1,))) + b_ref[0]


def _qkv(x2d, in_w, in_b):
    rb = 1024
    return _pcall(
        _qkv_body,
        grid=(3, S // rb),
        in_specs=[
            pl.BlockSpec((rb, D), lambda j, r: (r, 0)),
            pl.BlockSpec((D, D), lambda j, r: (j, 0)),
            pl.BlockSpec((1, 1, D), lambda j, r: (j, 0, 0)),
        ],
        out_specs=pl.BlockSpec((rb, D), lambda j, r: (r, j)),
        out_shape=jax.ShapeDtypeStruct((S, 3 * D), F32),
    )(x2d, in_w, in_b.reshape(3, 1, D))


_KC = 1024


def _attn_body(q_ref, k_ref, v_ref, o_ref):
    outs = []
    for i in (0, 1):
        q = q_ref[:, i * DH:(i + 1) * DH]
        k = k_ref[:, i * DH:(i + 1) * DH]
        v = v_ref[:, i * DH:(i + 1) * DH]
        s = _bdot(q, k, ((1,), (1,))) * 0.125
        m = None
        for c in range(0, S, _KC):
            sc = s[:, c:c + _KC]
            mc = jnp.max(sc, axis=1, keepdims=True)
            if m is None:
                m_new = mc
            else:
                m_new = jnp.maximum(m, mc)
            p = jnp.exp(sc - m_new)
            ov = _bdot(p, v[c:c + _KC, :], ((1,), (0,)))
            lc = jnp.sum(p, axis=1, keepdims=True)
            if m is None:
                o, l = ov, lc
            else:
                alpha = jnp.exp(m - m_new)
                o = o * alpha + ov
                l = l * alpha + lc
            m = m_new
        outs.append(o / l)
    o_ref[...] = jnp.concatenate(outs, axis=1)


def _attn(qkv):
    hp = NHEADS // 2
    return _pcall(
        _attn_body,
        grid=(hp, S // RB),
        in_specs=[
            pl.BlockSpec((RB, 2 * DH), lambda h, r: (r, h)),
            pl.BlockSpec((S, 2 * DH), lambda h, r: (0, hp + h)),
            pl.BlockSpec((S, 2 * DH), lambda h, r: (0, 2 * hp + h)),
        ],
        out_specs=pl.BlockSpec((RB, 2 * DH), lambda h, r: (r, h)),
        out_shape=jax.ShapeDtypeStruct((S, D), F32),
    )(qkv, qkv, qkv)


def _outln_body(o_ref, w_ref, b_ref, x_ref, g_ref, bb_ref, y_ref):
    t = _bdot(o_ref[...], w_ref[...], ((1,), (1,)))
    t = t + b_ref[...] + x_ref[...]
    m = jnp.mean(t, axis=1, keepdims=True)
    d = t - m
    v = jnp.mean(d * d, axis=1, keepdims=True)
    y_ref[...] = d / jnp.sqrt(v + 1e-5) * g_ref[...] + bb_ref[...]


def _outln(o, out_w, out_b, x2d, ln_g, ln_b):
    rb = 512
    return _pcall(
        _outln_body,
        grid=(S // rb,),
        in_specs=[
            pl.BlockSpec((rb, D), lambda r: (r, 0)),
            pl.BlockSpec((D, D), lambda r: (0, 0)),
            pl.BlockSpec((1, D), lambda r: (0, 0)),
            pl.BlockSpec((rb, D), lambda r: (r, 0)),
            pl.BlockSpec((1, D), lambda r: (0, 0)),
            pl.BlockSpec((1, D), lambda r: (0, 0)),
        ],
        out_specs=pl.BlockSpec((rb, D), lambda r: (r, 0)),
        out_shape=jax.ShapeDtypeStruct((S, D), F32),
    )(o, out_w, out_b.reshape(1, D), x2d, ln_g.reshape(1, D), ln_b.reshape(1, D))


def _router_body(x_ref, wg_ref, bg_ref, dst_ref, gat_ref):
    l = _bdot(wg_ref[...], x_ref[...], ((0,), (1,))) + bg_ref[...]
    iota_e = lax.broadcasted_iota(I32, (E, S), 0)
    m0 = jnp.max(l, axis=0, keepdims=True)
    e0 = jnp.min(jnp.where(l == m0, iota_e, E), axis=0, keepdims=True)
    lm = jnp.where(iota_e == e0, -jnp.inf, l)
    m1 = jnp.max(lm, axis=0, keepdims=True)
    e1 = jnp.min(jnp.where(lm == m1, iota_e, E), axis=0, keepdims=True)
    u = jnp.exp(m1 - m0)
    den = 1.0 + u
    g0 = 1.0 / den
    g1 = u / den
    w = jnp.where(iota_e == e0, g0, 0.0) + jnp.where(iota_e == e1, g1, 0.0)
    wb = lax.bitcast_convert_type(w, I32)

    def bis(_, carry):
        lo, hi = carry
        mid = (lo + hi) // 2
        c = jnp.sum((wb >= mid).astype(I32), axis=1, keepdims=True)
        take = c >= CAP
        return jnp.where(take, mid, lo), jnp.where(take, hi, mid)

    lo = jnp.zeros((E, 1), I32)
    hi = jnp.full((E, 1), 0x3F800001, I32)
    lo, hi = lax.fori_loop(0, 31, bis, (lo, hi))
    thr = lo
    n_gt = jnp.sum((wb > thr).astype(I32), axis=1, keepdims=True)
    quota = (CAP - n_gt).astype(F32)
    tie = (wb == thr) & (w > 0)
    tri = (lax.broadcasted_iota(I32, (S, S), 0)
           < lax.broadcasted_iota(I32, (S, S), 1)).astype(F32)
    tiepos = jnp.dot(tie.astype(F32), tri, preferred_element_type=F32)
    keep = (wb > thr) | (tie & (tiepos < quota))
    pos = jnp.dot(keep.astype(F32), tri, preferred_element_type=F32).astype(I32)

    def slot(e_sel, g_sel):
        onehot = iota_e == e_sel
        kept = jnp.sum(jnp.where(onehot & keep, 1, 0), axis=0, keepdims=True)
        p = jnp.sum(jnp.where(onehot, pos, 0), axis=0, keepdims=True)
        dst = jnp.where(kept > 0, e_sel * CAP + p, TRASH)
        g = jnp.where((kept > 0) & (g_sel > 0), g_sel, 0.0)
        return dst, g

    dst0, gg0 = slot(e0, g0)
    dst1, gg1 = slot(e1, g1)
    z_i = jnp.zeros((1, S), I32)
    z_f = jnp.zeros((1, S), F32)
    dst_ref[...] = jnp.concatenate([dst0, dst1, z_i, z_i, z_i, z_i, z_i, z_i], axis=0)
    gat_ref[...] = jnp.concatenate([gg0, gg1, z_f, z_f, z_f, z_f, z_f, z_f], axis=0)


def _router(x2d, wg, bg):
    return _pcall(
        _router_body,
        grid=(1,),
        in_specs=[
            pl.BlockSpec((S, D), lambda i: (0, 0)),
            pl.BlockSpec((D, E), lambda i: (0, 0)),
            pl.BlockSpec((E, 1), lambda i: (0, 0)),
        ],
        out_specs=[
            pl.BlockSpec((8, S), lambda i: (0, 0)),
            pl.BlockSpec((8, S), lambda i: (0, 0)),
        ],
        out_shape=[
            jax.ShapeDtypeStruct((8, S), I32),
            jax.ShapeDtypeStruct((8, S), F32),
        ],
    )(x2d, wg, bg.reshape(E, 1))


_SC_MESH = functools.partial(plsc.VectorSubcoreMesh,
                             core_axis_name="c", subcore_axis_name="s")


def _sc_dispatch(x2d, dst_all):
    @functools.partial(
        pl.kernel,
        out_type=jax.ShapeDtypeStruct((SLOT_ROWS, D), F32),
        mesh=_SC_MESH(),
        scratch_types=[
            pltpu.VMEM((128,), I32),
            pltpu.VMEM((128, D), F32),
            pltpu.SemaphoreType.DMA,
        ],
    )
    def body(x_hbm, dst_hbm, out_hbm, idx_v, rows_v, sem):
        wid = lax.axis_index("s") * 2 + lax.axis_index("c")
        base = wid * 128
        tok = lax.rem(wid, 16) * 128
        pltpu.sync_copy(dst_hbm.at[pl.ds(base, 128)], idx_v)
        pltpu.sync_copy(x_hbm.at[pl.ds(tok, 128)], rows_v)
        pltpu.async_copy(rows_v, out_hbm.at[idx_v], sem).wait()

    return body(x2d, dst_all)


def _sc_combine(he_all, dst_all):
    @functools.partial(
        pl.kernel,
        out_type=jax.ShapeDtypeStruct((2 * S, D), F32),
        mesh=_SC_MESH(),
        scratch_types=[
            pltpu.VMEM((128,), I32),
            pltpu.VMEM((128, D), F32),
            pltpu.SemaphoreType.DMA,
        ],
    )
    def body(he_hbm, dst_hbm, y_hbm, idx_v, rows_v, sem):
        wid = lax.axis_index("s") * 2 + lax.axis_index("c")
        base = wid * 128
        pltpu.sync_copy(dst_hbm.at[pl.ds(base, 128)], idx_v)
        pltpu.async_copy(he_hbm.at[idx_v], rows_v, sem).wait()
        pltpu.sync_copy(rows_v, y_hbm.at[pl.ds(base, 128)])

    return body(he_all, dst_all)


def _ffn_body(x_ref, w1_ref, b1_ref, w2_ref, b2_ref, o_ref):
    h = _bdot(x_ref[...], w1_ref[0], ((1,), (0,))) + b1_ref[0]
    h = jnp.maximum(h, 0.0)
    o_ref[...] = _bdot(h, w2_ref[0], ((1,), (0,))) + b2_ref[0]


def _ffn(xe, w1, b1, w2, b2):
    return _pcall(
        _ffn_body,
        grid=(E,),
        in_specs=[
            pl.BlockSpec((CAP, D), lambda e: (e, 0)),
            pl.BlockSpec((1, D, DHID), lambda e: (e, 0, 0)),
            pl.BlockSpec((1, 1, DHID), lambda e: (e, 0, 0)),
            pl.BlockSpec((1, DHID, D), lambda e: (e, 0, 0)),
            pl.BlockSpec((1, 1, D), lambda e: (e, 0, 0)),
        ],
        out_specs=pl.BlockSpec((CAP, D), lambda e: (e, 0)),
        out_shape=jax.ShapeDtypeStruct((SLOT_ROWS, D), F32),
    )(xe, w1, b1.reshape(E, 1, DHID), w2, b2.reshape(E, 1, D))


def _comb_body(x_ref, y0_ref, y1_ref, g_ref, o_ref):
    g0 = g_ref[:, 0:1]
    g1 = g_ref[:, 1:2]
    c0 = jnp.where(g0 > 0, g0 * y0_ref[...], 0.0)
    c1 = jnp.where(g1 > 0, g1 * y1_ref[...], 0.0)
    o_ref[...] = x_ref[...] + c0 + c1


def _combine_add(x2d, y01, gates_t):
    rb = 256
    return _pcall(
        _comb_body,
        grid=(S // rb,),
        in_specs=[
            pl.BlockSpec((rb, D), lambda r: (r, 0)),
            pl.BlockSpec((rb, D), lambda r: (r, 0)),
            pl.BlockSpec((rb, D), lambda r: (r + S // rb, 0)),
            pl.BlockSpec((rb, 8), lambda r: (r, 0)),
        ],
        out_specs=pl.BlockSpec((rb, D), lambda r: (r, 0)),
        out_shape=jax.ShapeDtypeStruct((S, D), F32),
    )(x2d, y01, y01, gates_t)


def kernel(x, params):
    s, b, d = x.shape
    x2d = x.reshape(s * b, d)
    for p in params:
        qkv = _qkv(x2d, p["in_w"], p["in_b"])
        o = _attn(qkv)
        xl = _outln(o, p["out_w"], p["out_b"], x2d, p["ln_g"], p["ln_b"])
        dst8, gat8 = _router(xl, p["wg"], p["bg"])
        dst_all = dst8[:2].reshape(2 * S)
        xe = _sc_dispatch(xl, dst_all)
        he = _ffn(xe, p["w1"], p["b1"], p["w2"], p["b2"])
        y01 = _sc_combine(he, dst_all)
        x2d = _combine_add(xl, y01, gat8.T)
    return x2d.reshape(s, b, d)


# speedup vs baseline: 5.5567x; 1.0211x over previous
import functools

import jax
import jax.numpy as jnp
from jax import lax
from jax.experimental import pallas as pl
from jax.experimental.pallas import tpu as pltpu
from jax.experimental.pallas import tpu_sc as plsc

F32 = jnp.float32
I32 = jnp.int32

NHEADS = 12
D = 768
DH = 64
DHID = 2048
E = 64
S = 2048
CAP = 128
NSLOT = E * CAP
TRASH = NSLOT
SLOT_ROWS = NSLOT + 8
RB = 512

_pcall = pl.pallas_call


BF16 = jnp.bfloat16


def _bdot(a, b, dims):
    return lax.dot_general(a.astype(BF16), b.astype(BF16), (dims, ((), ())),
                           preferred_element_type=F32)


def _qkv_body(x_ref, w_ref, b_ref, o_ref):
    o_ref[...] = _bdot(x_ref[...], w_ref[...], ((1,), (1,))) + b_ref[0]


def _qkv(x2d, in_w, in_b):
    rb = 1024
    return _pcall(
        _qkv_body,
        grid=(3, S // rb),
        in_specs=[
            pl.BlockSpec((rb, D), lambda j, r: (r, 0)),
            pl.BlockSpec((D, D), lambda j, r: (j, 0)),
            pl.BlockSpec((1, 1, D), lambda j, r: (j, 0, 0)),
        ],
        out_specs=pl.BlockSpec((rb, D), lambda j, r: (r, j)),
        out_shape=jax.ShapeDtypeStruct((S, 3 * D), F32),
    )(x2d, in_w, in_b.reshape(3, 1, D))


_KC = 1024


def _attn_body(q_ref, k_ref, v_ref, o_ref):
    outs = []
    for i in (0, 1):
        q = q_ref[:, i * DH:(i + 1) * DH]
        k = k_ref[:, i * DH:(i + 1) * DH]
        v = v_ref[:, i * DH:(i + 1) * DH]
        s = _bdot(q, k, ((1,), (1,))) * 0.125
        m = None
        for c in range(0, S, _KC):
            sc = s[:, c:c + _KC]
            mc = jnp.max(sc, axis=1, keepdims=True)
            if m is None:
                m_new = mc
            else:
                m_new = jnp.maximum(m, mc)
            p = jnp.exp(sc - m_new)
            ov = _bdot(p, v[c:c + _KC, :], ((1,), (0,)))
            lc = jnp.sum(p, axis=1, keepdims=True)
            if m is None:
                o, l = ov, lc
            else:
                alpha = jnp.exp(m - m_new)
                o = o * alpha + ov
                l = l * alpha + lc
            m = m_new
        outs.append(o / l)
    o_ref[...] = jnp.concatenate(outs, axis=1)


def _attn(qkv):
    hp = NHEADS // 2
    return _pcall(
        _attn_body,
        grid=(hp, S // RB),
        in_specs=[
            pl.BlockSpec((RB, 2 * DH), lambda h, r: (r, h)),
            pl.BlockSpec((S, 2 * DH), lambda h, r: (0, hp + h)),
            pl.BlockSpec((S, 2 * DH), lambda h, r: (0, 2 * hp + h)),
        ],
        out_specs=pl.BlockSpec((RB, 2 * DH), lambda h, r: (r, h)),
        out_shape=jax.ShapeDtypeStruct((S, D), F32),
    )(qkv, qkv, qkv)


def _outln_body(o_ref, w_ref, b_ref, x_ref, g_ref, bb_ref, y_ref):
    t = _bdot(o_ref[...], w_ref[...], ((1,), (1,)))
    t = t + b_ref[...] + x_ref[...]
    m = jnp.mean(t, axis=1, keepdims=True)
    d = t - m
    v = jnp.mean(d * d, axis=1, keepdims=True)
    y_ref[...] = d / jnp.sqrt(v + 1e-5) * g_ref[...] + bb_ref[...]


def _outln(o, out_w, out_b, x2d, ln_g, ln_b):
    rb = 512
    return _pcall(
        _outln_body,
        grid=(S // rb,),
        in_specs=[
            pl.BlockSpec((rb, D), lambda r: (r, 0)),
            pl.BlockSpec((D, D), lambda r: (0, 0)),
            pl.BlockSpec((1, D), lambda r: (0, 0)),
            pl.BlockSpec((rb, D), lambda r: (r, 0)),
            pl.BlockSpec((1, D), lambda r: (0, 0)),
            pl.BlockSpec((1, D), lambda r: (0, 0)),
        ],
        out_specs=pl.BlockSpec((rb, D), lambda r: (r, 0)),
        out_shape=jax.ShapeDtypeStruct((S, D), F32),
    )(o, out_w, out_b.reshape(1, D), x2d, ln_g.reshape(1, D), ln_b.reshape(1, D))


def _router_body(x_ref, wg_ref, bg_ref, dst_ref, gat_ref):
    l = _bdot(wg_ref[...], x_ref[...], ((0,), (1,))) + bg_ref[...]
    iota_e = lax.broadcasted_iota(I32, (E, S), 0)
    m0 = jnp.max(l, axis=0, keepdims=True)
    e0 = jnp.min(jnp.where(l == m0, iota_e, E), axis=0, keepdims=True)
    lm = jnp.where(iota_e == e0, -jnp.inf, l)
    m1 = jnp.max(lm, axis=0, keepdims=True)
    e1 = jnp.min(jnp.where(lm == m1, iota_e, E), axis=0, keepdims=True)
    u = jnp.exp(m1 - m0)
    den = 1.0 + u
    g0 = 1.0 / den
    g1 = u / den
    w = jnp.where(iota_e == e0, g0, 0.0) + jnp.where(iota_e == e1, g1, 0.0)
    wb = lax.bitcast_convert_type(w, I32)

    def bis(_, carry):
        lo, hi = carry
        mid = (lo + hi) // 2
        c = jnp.sum((wb >= mid).astype(I32), axis=1, keepdims=True)
        take = c >= CAP
        return jnp.where(take, mid, lo), jnp.where(take, hi, mid)

    lo = jnp.zeros((E, 1), I32)
    hi = jnp.full((E, 1), 0x3F800001, I32)
    lo, hi = lax.fori_loop(0, 31, bis, (lo, hi))
    thr = lo
    n_gt = jnp.sum((wb > thr).astype(I32), axis=1, keepdims=True)
    quota = (CAP - n_gt).astype(F32)
    tie = (wb == thr) & (w > 0)
    tri = (lax.broadcasted_iota(I32, (S, S), 0)
           < lax.broadcasted_iota(I32, (S, S), 1)).astype(F32)
    tiepos = jnp.dot(tie.astype(F32), tri, preferred_element_type=F32)
    keep = (wb > thr) | (tie & (tiepos < quota))
    pos = jnp.dot(keep.astype(F32), tri, preferred_element_type=F32).astype(I32)

    def slot(e_sel, g_sel):
        onehot = iota_e == e_sel
        kept = jnp.sum(jnp.where(onehot & keep, 1, 0), axis=0, keepdims=True)
        p = jnp.sum(jnp.where(onehot, pos, 0), axis=0, keepdims=True)
        dst = jnp.where(kept > 0, e_sel * CAP + p, TRASH)
        g = jnp.where((kept > 0) & (g_sel > 0), g_sel, 0.0)
        return dst, g

    dst0, gg0 = slot(e0, g0)
    dst1, gg1 = slot(e1, g1)
    z_i = jnp.zeros((1, S), I32)
    z_f = jnp.zeros((1, S), F32)
    dst_ref[...] = jnp.concatenate([dst0, dst1, z_i, z_i, z_i, z_i, z_i, z_i], axis=0)
    gat_ref[...] = jnp.transpose(
        jnp.concatenate([gg0, gg1, z_f, z_f, z_f, z_f, z_f, z_f], axis=0))


def _router(x2d, wg, bg):
    return _pcall(
        _router_body,
        grid=(1,),
        in_specs=[
            pl.BlockSpec((S, D), lambda i: (0, 0)),
            pl.BlockSpec((D, E), lambda i: (0, 0)),
            pl.BlockSpec((E, 1), lambda i: (0, 0)),
        ],
        out_specs=[
            pl.BlockSpec((8, S), lambda i: (0, 0)),
            pl.BlockSpec((S, 8), lambda i: (0, 0)),
        ],
        out_shape=[
            jax.ShapeDtypeStruct((8, S), I32),
            jax.ShapeDtypeStruct((S, 8), F32),
        ],
    )(x2d, wg, bg.reshape(E, 1))


_SC_MESH = functools.partial(plsc.VectorSubcoreMesh,
                             core_axis_name="c", subcore_axis_name="s")


def _sc_dispatch(x2d, dst_all):
    @functools.partial(
        pl.kernel,
        out_type=jax.ShapeDtypeStruct((SLOT_ROWS, D), F32),
        mesh=_SC_MESH(),
        scratch_types=[
            pltpu.VMEM((128,), I32),
            pltpu.VMEM((128, D), F32),
            pltpu.SemaphoreType.DMA,
        ],
    )
    def body(x_hbm, dst_hbm, out_hbm, idx_v, rows_v, sem):
        wid = lax.axis_index("s") * 2 + lax.axis_index("c")
        slot = wid // 16
        tok = lax.rem(wid, 16) * 128
        pltpu.sync_copy(dst_hbm.at[slot, pl.ds(tok, 128)], idx_v)
        pltpu.sync_copy(x_hbm.at[pl.ds(tok, 128)], rows_v)
        pltpu.async_copy(rows_v, out_hbm.at[idx_v], sem).wait()

    return body(x2d, dst_all)


def _sc_combine(he_all, dst_all):
    @functools.partial(
        pl.kernel,
        out_type=jax.ShapeDtypeStruct((2 * S, D), F32),
        mesh=_SC_MESH(),
        scratch_types=[
            pltpu.VMEM((128,), I32),
            pltpu.VMEM((128, D), F32),
            pltpu.SemaphoreType.DMA,
        ],
    )
    def body(he_hbm, dst_hbm, y_hbm, idx_v, rows_v, sem):
        wid = lax.axis_index("s") * 2 + lax.axis_index("c")
        slot = wid // 16
        tok = lax.rem(wid, 16) * 128
        pltpu.sync_copy(dst_hbm.at[slot, pl.ds(tok, 128)], idx_v)
        pltpu.async_copy(he_hbm.at[idx_v], rows_v, sem).wait()
        pltpu.sync_copy(rows_v, y_hbm.at[pl.ds(slot * S + tok, 128)])

    return body(he_all, dst_all)


def _ffn_body(x_ref, w1_ref, b1_ref, w2_ref, b2_ref, o_ref):
    h = _bdot(x_ref[...], w1_ref[0], ((1,), (0,))) + b1_ref[0]
    h = jnp.maximum(h, 0.0)
    o_ref[...] = _bdot(h, w2_ref[0], ((1,), (0,))) + b2_ref[0]


def _ffn(xe, w1, b1, w2, b2):
    return _pcall(
        _ffn_body,
        grid=(E,),
        in_specs=[
            pl.BlockSpec((CAP, D), lambda e: (e, 0)),
            pl.BlockSpec((1, D, DHID), lambda e: (e, 0, 0)),
            pl.BlockSpec((1, 1, DHID), lambda e: (e, 0, 0)),
            pl.BlockSpec((1, DHID, D), lambda e: (e, 0, 0)),
            pl.BlockSpec((1, 1, D), lambda e: (e, 0, 0)),
        ],
        out_specs=pl.BlockSpec((CAP, D), lambda e: (e, 0)),
        out_shape=jax.ShapeDtypeStruct((SLOT_ROWS, D), F32),
    )(xe, w1, b1.reshape(E, 1, DHID), w2, b2.reshape(E, 1, D))


def _comb_body(x_ref, y0_ref, y1_ref, g_ref, o_ref):
    g0 = g_ref[:, 0:1]
    g1 = g_ref[:, 1:2]
    c0 = jnp.where(g0 > 0, g0 * y0_ref[...], 0.0)
    c1 = jnp.where(g1 > 0, g1 * y1_ref[...], 0.0)
    o_ref[...] = x_ref[...] + c0 + c1


def _combine_add(x2d, y01, gates_t):
    rb = 256
    return _pcall(
        _comb_body,
        grid=(S // rb,),
        in_specs=[
            pl.BlockSpec((rb, D), lambda r: (r, 0)),
            pl.BlockSpec((rb, D), lambda r: (r, 0)),
            pl.BlockSpec((rb, D), lambda r: (r + S // rb, 0)),
            pl.BlockSpec((rb, 8), lambda r: (r, 0)),
        ],
        out_specs=pl.BlockSpec((rb, D), lambda r: (r, 0)),
        out_shape=jax.ShapeDtypeStruct((S, D), F32),
    )(x2d, y01, y01, gates_t)


def kernel(x, params):
    s, b, d = x.shape
    x2d = x.reshape(s * b, d)
    for p in params:
        qkv = _qkv(x2d, p["in_w"], p["in_b"])
        o = _attn(qkv)
        xl = _outln(o, p["out_w"], p["out_b"], x2d, p["ln_g"], p["ln_b"])
        dst8, gatT = _router(xl, p["wg"], p["bg"])
        xe = _sc_dispatch(xl, dst8)
        he = _ffn(xe, p["w1"], p["b1"], p["w2"], p["b2"])
        y01 = _sc_combine(he, dst8)
        x2d = _combine_add(xl, y01, gatT)
    return x2d.reshape(s, b, d)
